# baseline (speedup 1.0000x reference)
import functools

import jax
import jax.numpy as jnp
from jax import lax
from jax.experimental import pallas as pl
from jax.experimental.pallas import tpu as pltpu
from jax.experimental.pallas import tpu_sc as plsc

_B = 16384
_F = 16
_L = 16
_NC = 2
_NS = 16
_NW = _NC * _NS
_BPW = _B // _NW
_CHUNK = 128
_NCHUNK = _BPW // _CHUNK

_mesh = plsc.VectorSubcoreMesh(
    core_axis_name="c", subcore_axis_name="s", num_cores=_NC, num_subcores=_NS
)


@functools.partial(
    pl.kernel,
    out_type=jax.ShapeDtypeStruct((_B,), jnp.float32),
    mesh=_mesh,
    compiler_params=pltpu.CompilerParams(
        needs_layout_passes=False, use_tc_tiling_on_sc=False
    ),
    scratch_types=[
        pltpu.VMEM((_NCHUNK, _CHUNK), jnp.int32),
        pltpu.VMEM((_NCHUNK, _CHUNK), jnp.int32),
        pltpu.VMEM((_BPW, _F), jnp.float32),
        pltpu.VMEM((_BPW, _F), jnp.float32),
        pltpu.VMEM((_BPW,), jnp.float32),
        pltpu.SemaphoreType.DMA,
    ],
)
def _mf_kernel(user_hbm, item_hbm, uf_hbm, if_hbm, out_hbm,
               uidx_v, iidx_v, urows_v, irows_v, out_v, sem):
    wid = lax.axis_index("s") * _NC + lax.axis_index("c")
    base = wid * _BPW

    cbase = wid * _NCHUNK
    pltpu.sync_copy(user_hbm.at[pl.ds(cbase, _NCHUNK)], uidx_v)
    pltpu.sync_copy(item_hbm.at[pl.ds(cbase, _NCHUNK)], iidx_v)

    for c in range(_NCHUNK):
        for k in range(_CHUNK // _L):
            sl = pl.ds(k * _L, _L)
            uidx_v[c, sl] = uidx_v[c, sl] - 1
            iidx_v[c, sl] = iidx_v[c, sl] - 1

    copies = []
    for c in range(_NCHUNK):
        dst = pl.ds(c * _CHUNK, _CHUNK)
        copies.append(pltpu.async_copy(uf_hbm.at[uidx_v.at[c]], urows_v.at[dst], sem))
        copies.append(pltpu.async_copy(if_hbm.at[iidx_v.at[c]], irows_v.at[dst], sem))
    for cp in copies:
        cp.wait()

    iota16 = lax.iota(jnp.int32, _L)

    def body(g, carry):
        rows = g * _L + iota16
        acc = jnp.zeros((_L,), jnp.float32)
        for f in range(_F):
            cols = jnp.full((_L,), f, jnp.int32)
            u = plsc.load_gather(urows_v, [rows, cols])
            it = plsc.load_gather(irows_v, [rows, cols])
            acc = acc + u * it
        out_v[pl.ds(pl.multiple_of(g * _L, _L), _L)] = acc * 5.0
        return carry

    lax.fori_loop(0, _BPW // _L, body, 0)

    pltpu.sync_copy(out_v, out_hbm.at[pl.ds(base, _BPW)])


def kernel(user, item, user_factors, item_factors):
    user2 = user.reshape(_B // _CHUNK, _CHUNK)
    item2 = item.reshape(_B // _CHUNK, _CHUNK)
    return _mf_kernel(user2, item2, user_factors, item_factors)


# speedup vs baseline: 5.7805x; 5.7805x over previous
import functools

import jax
import jax.numpy as jnp
from jax import lax
from jax.experimental import pallas as pl
from jax.experimental.pallas import tpu as pltpu
from jax.experimental.pallas import tpu_sc as plsc

_B = 16384
_F = 16
_L = 16
_NC = 2
_NS = 16
_NW = _NC * _NS
_BPW = _B // _NW
_G = 16

_mesh = plsc.VectorSubcoreMesh(
    core_axis_name="c", subcore_axis_name="s", num_cores=_NC, num_subcores=_NS
)


@functools.partial(
    pl.kernel,
    out_type=jax.ShapeDtypeStruct((_B,), jnp.float32),
    mesh=_mesh,
    compiler_params=pltpu.CompilerParams(
        needs_layout_passes=False, use_tc_tiling_on_sc=True
    ),
    scratch_types=[
        pltpu.VMEM((_BPW,), jnp.int32),
        pltpu.VMEM((_BPW,), jnp.int32),
        pltpu.VMEM((_G, _F, 128), jnp.float32),
        pltpu.VMEM((_G, _F, 128), jnp.float32),
        pltpu.VMEM((_G * _L,), jnp.float32),
        pltpu.VMEM((_BPW,), jnp.float32),
        pltpu.SemaphoreType.DMA,
    ],
)
def _mf_kernel(user_hbm, item_hbm, uft_hbm, ift_hbm, out_hbm,
               uidx_v, iidx_v, ubufs_v, ibufs_v, prod_v,
               out_v, sem):
    wid = lax.axis_index("s") * _NC + lax.axis_index("c")
    base = wid * _BPW

    pltpu.sync_copy(user_hbm.at[pl.ds(base, _BPW)], uidx_v)
    pltpu.sync_copy(item_hbm.at[pl.ds(base, _BPW)], iidx_v)

    iota16 = lax.iota(jnp.int32, _L)

    def group(g, carry):
        gsl = pl.ds(pl.multiple_of(g * _G, _G), _G)
        uvec = uidx_v[gsl] - 1
        ivec = iidx_v[gsl] - 1
        copies = []
        for k in range(_G):
            u = uvec[k]
            i = ivec[k]
            ua = pl.multiple_of((u // 128) * 128, 128)
            ia = pl.multiple_of((i // 128) * 128, 128)
            copies.append(pltpu.async_copy(
                uft_hbm.at[:, pl.ds(ua, 128)], ubufs_v.at[k], sem))
            copies.append(pltpu.async_copy(
                ift_hbm.at[:, pl.ds(ia, 128)], ibufs_v.at[k], sem))
        for cp in copies:
            cp.wait()
        for k in range(_G):
            lu = jnp.full((_L,), uvec[k] % 128, jnp.int32)
            li = jnp.full((_L,), ivec[k] % 128, jnp.int32)
            ucol = plsc.load_gather(ubufs_v.at[k], [iota16, lu])
            icol = plsc.load_gather(ibufs_v.at[k], [iota16, li])
            plsc.store_scatter(prod_v, [k * _L + iota16], ucol * icol)
        acc = plsc.load_gather(prod_v, [iota16 * _L])
        for f in range(1, _F):
            acc = acc + plsc.load_gather(prod_v, [iota16 * _L + f])
        plsc.store_scatter(out_v, [g * _L + iota16], acc * 5.0)
        return carry

    lax.fori_loop(0, _BPW // _G, group, 0)

    pltpu.sync_copy(out_v, out_hbm.at[pl.ds(base, _BPW)])


def kernel(user, item, user_factors, item_factors):
    return _mf_kernel(user, item, user_factors.T, item_factors.T)


# speedup vs baseline: 5.8946x; 1.0197x over previous
import functools

import jax
import jax.numpy as jnp
from jax import lax
from jax.experimental import pallas as pl
from jax.experimental.pallas import tpu as pltpu
from jax.experimental.pallas import tpu_sc as plsc

_B = 16384
_F = 16
_L = 16
_NC = 2
_NS = 16
_NW = _NC * _NS
_BPW = _B // _NW
_G = 8
_NT = _BPW // (2 * _G)

_mesh = plsc.VectorSubcoreMesh(
    core_axis_name="c", subcore_axis_name="s", num_cores=_NC, num_subcores=_NS
)


@functools.partial(
    pl.kernel,
    out_type=jax.ShapeDtypeStruct((_B,), jnp.float32),
    mesh=_mesh,
    compiler_params=pltpu.CompilerParams(
        needs_layout_passes=False, use_tc_tiling_on_sc=True
    ),
    scratch_types=[
        pltpu.VMEM((_BPW,), jnp.int32),
        pltpu.VMEM((_BPW,), jnp.int32),
        pltpu.VMEM((_G, _F, 128), jnp.float32),
        pltpu.VMEM((_G, _F, 128), jnp.float32),
        pltpu.VMEM((_G, _F, 128), jnp.float32),
        pltpu.VMEM((_G, _F, 128), jnp.float32),
        pltpu.VMEM((2 * _G * _L,), jnp.float32),
        pltpu.VMEM((_BPW,), jnp.float32),
        pltpu.SemaphoreType.DMA,
        pltpu.SemaphoreType.DMA,
    ],
)
def _mf_kernel(user_hbm, item_hbm, uft_hbm, ift_hbm, out_hbm,
               uidx_v, iidx_v, ubufs_a, ibufs_a, ubufs_b, ibufs_b,
               prod_v, out_v, sem_a, sem_b):
    wid = lax.axis_index("s") * _NC + lax.axis_index("c")
    base = wid * _BPW

    pltpu.sync_copy(user_hbm.at[pl.ds(base, _BPW)], uidx_v)
    pltpu.sync_copy(item_hbm.at[pl.ds(base, _BPW)], iidx_v)

    iota16 = lax.iota(jnp.int32, _L)

    def load_ids(t):
        gsl = pl.ds(pl.multiple_of(t * 2 * _G, _L), _L)
        return uidx_v[gsl] - 1, iidx_v[gsl] - 1

    def issue(uvec, ivec, half, ubufs, ibufs, sem):
        copies = []
        for k in range(_G):
            u = uvec[half * _G + k]
            i = ivec[half * _G + k]
            ua = pl.multiple_of((u // 128) * 128, 128)
            ia = pl.multiple_of((i // 128) * 128, 128)
            copies.append(pltpu.async_copy(
                uft_hbm.at[:, pl.ds(ua, 128)], ubufs.at[k], sem))
            copies.append(pltpu.async_copy(
                ift_hbm.at[:, pl.ds(ia, 128)], ibufs.at[k], sem))
        return copies

    def extract(uvec, ivec, half, ubufs, ibufs):
        for k in range(_G):
            lu = jnp.full((_L,), uvec[half * _G + k] % 128, jnp.int32)
            li = jnp.full((_L,), ivec[half * _G + k] % 128, jnp.int32)
            ucol = plsc.load_gather(ubufs.at[k], [iota16, lu])
            icol = plsc.load_gather(ibufs.at[k], [iota16, li])
            plsc.store_scatter(
                prod_v, [(half * _G + k) * _L + iota16], ucol * icol)

    def drain(ubufs, ibufs, sem):
        for k in range(_G):
            pltpu.make_async_copy(
                uft_hbm.at[:, pl.ds(0, 128)], ubufs.at[k], sem).wait()
            pltpu.make_async_copy(
                ift_hbm.at[:, pl.ds(0, 128)], ibufs.at[k], sem).wait()

    uv0, iv0 = load_ids(0)
    issue(uv0, iv0, 0, ubufs_a, ibufs_a, sem_a)
    issue(uv0, iv0, 1, ubufs_b, ibufs_b, sem_b)

    def step(t, carry):
        uvec, ivec = load_ids(t)
        nxt = jnp.minimum(t + 1, _NT - 1)
        uvn, ivn = load_ids(nxt)

        drain(ubufs_a, ibufs_a, sem_a)
        extract(uvec, ivec, 0, ubufs_a, ibufs_a)
        issue(uvn, ivn, 0, ubufs_a, ibufs_a, sem_a)

        drain(ubufs_b, ibufs_b, sem_b)
        extract(uvec, ivec, 1, ubufs_b, ibufs_b)
        issue(uvn, ivn, 1, ubufs_b, ibufs_b, sem_b)

        acc = plsc.load_gather(prod_v, [iota16 * _L])
        for f in range(1, _F):
            acc = acc + plsc.load_gather(prod_v, [iota16 * _L + f])
        plsc.store_scatter(out_v, [t * _L + iota16], acc * 5.0)
        return carry

    lax.fori_loop(0, _NT, step, 0)

    drain(ubufs_a, ibufs_a, sem_a)
    drain(ubufs_b, ibufs_b, sem_b)

    pltpu.sync_copy(out_v, out_hbm.at[pl.ds(base, _BPW)])


def kernel(user, item, user_factors, item_factors):
    return _mf_kernel(user, item, user_factors.T, item_factors.T)
